# baseline (speedup 1.0000x reference)
import functools

import jax
import jax.numpy as jnp
from jax.experimental import pallas as pl
from jax.experimental.pallas import tpu as pltpu

_NBUF = 3


def _pick_bm(n):
    for bm in (400, 512, 256, 200, 128, 80, 64, 40, 32, 16, 8):
        if n % bm == 0:
            return bm
    return n


def _body(feat_ref, w_ref, adj_hbm, out_ref, h_ref, buf, sem, *, bm, grid_n):
    i = pl.program_id(0)

    @pl.when(i == 0)
    def _():
        for s in range(min(_NBUF, grid_n)):
            pltpu.make_async_copy(
                adj_hbm.at[pl.ds(s * bm, bm), :], buf.at[s], sem.at[s]).start()
        h_ref[...] = 0.5 * jnp.dot(
            feat_ref[...], w_ref[...], preferred_element_type=jnp.float32)

    slot = jax.lax.rem(i, _NBUF)
    pltpu.make_async_copy(
        adj_hbm.at[pl.ds(i * bm, bm), :], buf.at[slot], sem.at[slot]).wait()
    h_rows = h_ref[pl.ds(i * bm, bm), :]
    out_ref[...] = h_rows + jnp.dot(
        buf[slot], h_ref[...], preferred_element_type=jnp.float32)

    @pl.when(i + _NBUF < grid_n)
    def _():
        pltpu.make_async_copy(
            adj_hbm.at[pl.ds((i + _NBUF) * bm, bm), :],
            buf.at[slot], sem.at[slot]).start()


@jax.jit
def kernel(feature, adj, W):
    n, d = feature.shape
    e = W.shape[1]
    bm = _pick_bm(n)
    grid_n = n // bm
    return pl.pallas_call(
        functools.partial(_body, bm=bm, grid_n=grid_n),
        grid=(grid_n,),
        in_specs=[
            pl.BlockSpec((n, d), lambda i: (0, 0)),
            pl.BlockSpec((d, e), lambda i: (0, 0)),
            pl.BlockSpec(memory_space=pl.ANY),
        ],
        out_specs=pl.BlockSpec((bm, e), lambda i: (i, 0)),
        out_shape=jax.ShapeDtypeStruct((n, e), jnp.float32),
        scratch_shapes=[
            pltpu.VMEM((n, e), jnp.float32),
            pltpu.VMEM((_NBUF, bm, n), jnp.float32),
            pltpu.SemaphoreType.DMA((_NBUF,)),
        ],
        compiler_params=pltpu.CompilerParams(
            dimension_semantics=("arbitrary",),
        ),
    )(feature, W, adj)


# speedup vs baseline: 1.0491x; 1.0491x over previous
import functools

import jax
import jax.numpy as jnp
from jax.experimental import pallas as pl
from jax.experimental.pallas import tpu as pltpu

_NBUF = 2


def _pick_bm(n):
    for bm in (200, 512, 400, 256, 128, 80, 64, 40, 32, 16, 8):
        if n % bm == 0:
            return bm
    return n


def _body(feat_ref, w_ref, adj_hbm, out_ref, h_ref, buf, sem, *, bm, grid_n):
    i = pl.program_id(0)

    @pl.when(i == 0)
    def _():
        for s in range(min(_NBUF, grid_n)):
            pltpu.make_async_copy(
                adj_hbm.at[pl.ds(s * bm, bm), :], buf.at[s], sem.at[s]).start()
        h_ref[...] = 0.5 * jnp.dot(
            feat_ref[...], w_ref[...], preferred_element_type=jnp.float32)

    slot = jax.lax.rem(i, _NBUF)
    pltpu.make_async_copy(
        adj_hbm.at[pl.ds(i * bm, bm), :], buf.at[slot], sem.at[slot]).wait()
    h_rows = h_ref[pl.ds(i * bm, bm), :]
    out_ref[...] = h_rows + jnp.dot(
        buf[slot], h_ref[...], preferred_element_type=jnp.float32)

    @pl.when(i + _NBUF < grid_n)
    def _():
        pltpu.make_async_copy(
            adj_hbm.at[pl.ds((i + _NBUF) * bm, bm), :],
            buf.at[slot], sem.at[slot]).start()


@jax.jit
def kernel(feature, adj, W):
    n, d = feature.shape
    e = W.shape[1]
    bm = _pick_bm(n)
    grid_n = n // bm
    return pl.pallas_call(
        functools.partial(_body, bm=bm, grid_n=grid_n),
        grid=(grid_n,),
        in_specs=[
            pl.BlockSpec((n, d), lambda i: (0, 0)),
            pl.BlockSpec((d, e), lambda i: (0, 0)),
            pl.BlockSpec(memory_space=pl.ANY),
        ],
        out_specs=pl.BlockSpec((bm, e), lambda i: (i, 0)),
        out_shape=jax.ShapeDtypeStruct((n, e), jnp.float32),
        scratch_shapes=[
            pltpu.VMEM((n, e), jnp.float32),
            pltpu.VMEM((_NBUF, bm, n), jnp.float32),
            pltpu.SemaphoreType.DMA((_NBUF,)),
        ],
        compiler_params=pltpu.CompilerParams(
            dimension_semantics=("arbitrary",),
        ),
    )(feature, W, adj)
